# baseline (speedup 1.0000x reference)
import functools
import math

import jax
import jax.numpy as jnp
from jax import lax
from jax.experimental import pallas as pl
from jax.experimental.pallas import tpu as pltpu
from jax.experimental.pallas import tpu_sc as plsc

D_MODEL = 64
SCALE = math.sqrt(D_MODEL)


@functools.lru_cache(maxsize=None)
def _make_sc_kernel(B: int, V: int, D: int):
    info = plsc.get_sparse_core_info()
    NC, NS, L = info.num_cores, info.num_subcores, info.num_lanes
    NW = NC * NS
    assert B % NW == 0 and D % L == 0
    b_per_w = B // NW
    C = 512
    assert b_per_w % C == 0
    n_chunks = b_per_w // C
    vecs_per_row = D // L
    mesh = plsc.VectorSubcoreMesh(core_axis_name="c", subcore_axis_name="s")

    @functools.partial(
        pl.kernel,
        mesh=mesh,
        out_type=jax.ShapeDtypeStruct((B, D), jnp.float32),
        compiler_params=pltpu.CompilerParams(use_tc_tiling_on_sc=False),
        scratch_types=[
            pltpu.VMEM((C,), jnp.int32),
            pltpu.VMEM((C, D), jnp.float32),
            pltpu.SemaphoreType.DMA,
        ],
    )
    def k(x_hbm, table_hbm, out_hbm, idx_v, rows_v, sem):
        wid = lax.axis_index("s") * NC + lax.axis_index("c")
        base = wid * b_per_w

        @pl.loop(0, n_chunks)
        def _chunk(g):
            off = base + g * C
            pltpu.sync_copy(x_hbm.at[pl.ds(off, C)], idx_v)
            pltpu.async_copy(table_hbm.at[idx_v], rows_v, sem).wait()

            @plsc.parallel_loop(0, C)
            def _scale(r):
                for d in range(vecs_per_row):
                    sl = pl.ds(d * L, L)
                    rows_v[r, sl] = rows_v[r, sl] * SCALE

            pltpu.sync_copy(rows_v, out_hbm.at[pl.ds(off, C)])

    return k


def kernel(x, table):
    B = x.shape[0] * x.shape[1]
    V, D = table.shape
    xf = x.reshape(B)
    out = _make_sc_kernel(B, V, D)(xf, table)
    return out.reshape(x.shape[0], x.shape[1], D)


# speedup vs baseline: 1.1114x; 1.1114x over previous
import functools
import math

import jax
import jax.numpy as jnp
from jax import lax
from jax.experimental import pallas as pl
from jax.experimental.pallas import tpu as pltpu
from jax.experimental.pallas import tpu_sc as plsc

D_MODEL = 64
SCALE = math.sqrt(D_MODEL)


@functools.lru_cache(maxsize=None)
def _make_sc_kernel(B: int, V: int, D: int):
    info = plsc.get_sparse_core_info()
    NC, NS, L = info.num_cores, info.num_subcores, info.num_lanes
    NW = NC * NS
    assert B % NW == 0 and D % L == 0
    b_per_w = B // NW
    C = 256
    NB = 4
    AH = 2
    assert b_per_w % C == 0
    n_chunks = b_per_w // C
    assert n_chunks % NB == 0 and n_chunks > NB
    vecs_per_row = D // L
    mesh = plsc.VectorSubcoreMesh(core_axis_name="c", subcore_axis_name="s")

    @functools.partial(
        pl.kernel,
        mesh=mesh,
        out_type=jax.ShapeDtypeStruct((B, D), jnp.float32),
        compiler_params=pltpu.CompilerParams(use_tc_tiling_on_sc=False),
        scratch_types=(
            [pltpu.VMEM((b_per_w,), jnp.int32)]
            + [pltpu.VMEM((C, D), jnp.float32) for _ in range(NB)]
            + [pltpu.SemaphoreType.DMA for _ in range(2 * NB)]
        ),
    )
    def k(x_hbm, table_hbm, out_hbm, *scr):
        idx_all = scr[0]
        rows = scr[1 : 1 + NB]
        gsem = scr[1 + NB : 1 + 2 * NB]
        ssem = scr[1 + 2 * NB : 1 + 3 * NB]
        wid = lax.axis_index("s") * NC + lax.axis_index("c")
        base = wid * b_per_w

        pltpu.sync_copy(x_hbm.at[pl.ds(base, b_per_w)], idx_all)

        def start_gather(c, b):
            pltpu.async_copy(
                table_hbm.at[idx_all.at[pl.ds(c * C, C)]], rows[b], gsem[b]
            )

        for c in range(AH):
            start_gather(c, c % NB)

        @pl.loop(0, n_chunks, step=NB)
        def _grp(g):
            for b in range(NB):
                c = g + b
                pre = c + AH
                b2 = (b + AH) % NB

                @pl.when(pre < n_chunks)
                def _():
                    @pl.when(pre >= NB)
                    def _():
                        pltpu.make_async_copy(
                            rows[b2], out_hbm.at[pl.ds(0, C)], ssem[b2]
                        ).wait()

                    start_gather(pre, b2)

                pltpu.make_async_copy(
                    table_hbm.at[idx_all.at[pl.ds(0, C)]], rows[b], gsem[b]
                ).wait()

                @plsc.parallel_loop(0, C, unroll=2)
                def _scale(r):
                    for d in range(vecs_per_row):
                        sl = pl.ds(d * L, L)
                        rows[b][r, sl] = rows[b][r, sl] * SCALE

                pltpu.async_copy(
                    rows[b], out_hbm.at[pl.ds(base + c * C, C)], ssem[b]
                )

        for b in range(NB):
            pltpu.make_async_copy(
                rows[b], out_hbm.at[pl.ds(0, C)], ssem[b]
            ).wait()

    return k


def kernel(x, table):
    B = x.shape[0] * x.shape[1]
    V, D = table.shape
    xf = x.reshape(B)
    out = _make_sc_kernel(B, V, D)(xf, table)
    return out.reshape(x.shape[0], x.shape[1], D)
